# baseline (speedup 1.0000x reference)
import functools

import jax
import jax.numpy as jnp
from jax import lax
from jax.experimental import pallas as pl
from jax.experimental.pallas import tpu as pltpu
from jax.experimental.pallas import tpu_sc as plsc

_KEEP_RATIO = 0.05
_INT_MIN = -(2 ** 31)
_NC, _NS, _L = 2, 16, 16
_NW = _NC * _NS


def _sc_thresholds_body(x_hbm, thr_hbm, row_v, ukey_v, hist_v, out_v, sem,
                        *, rows_per_w: int, d: int, k: int, unroll: int):
    cid = lax.axis_index("c")
    sid = lax.axis_index("s")
    w = sid * _NC + cid
    lanes = lax.iota(jnp.int32, _L)
    zeros16 = jnp.zeros((_L,), jnp.int32)
    nvec = d // _L

    def scan_hist(krem):
        above = jnp.int32(0)
        bstar = jnp.int32(0)
        above_b = jnp.int32(0)
        found = jnp.bool_(False)
        for j in range(15, -1, -1):
            t = hist_v[pl.ds(j * _L, _L)]
            rt = lax.rev(t, (0,))
            cs = plsc.cumsum(rt)
            s_cum = above + cs
            below = jnp.sum((s_cum < krem).astype(jnp.int32))
            hit = jnp.logical_and(jnp.logical_not(found), below < _L)
            bin_here = jnp.int32(j * _L + (_L - 1)) - below
            abv_here = above + jnp.sum(
                jnp.where(lanes < below, rt, jnp.int32(0)))
            bstar = jnp.where(hit, bin_here, bstar)
            above_b = jnp.where(hit, abv_here, above_b)
            found = jnp.logical_or(found, below < _L)
            above = above + jnp.sum(t)
        return bstar, above_b

    def zero_hist():
        for j in range(256 // _L):
            hist_v[pl.ds(j * _L, _L)] = zeros16

    nxt = pltpu.async_copy(x_hbm.at[w * rows_per_w], row_v.at[0], sem)
    for r in range(rows_per_w):
        row = w * rows_per_w + r
        nxt.wait()
        if r + 1 < rows_per_w:
            nxt = pltpu.async_copy(
                x_hbm.at[row + 1], row_v.at[(r + 1) % 2], sem)
        buf = r % 2

        zero_hist()

        @plsc.parallel_loop(0, nvec, unroll=unroll)
        def _(i):
            s = row_v[buf, pl.ds(i * _L, _L)]
            uk = s ^ ((s >> 31) | jnp.int32(_INT_MIN))
            ukey_v[pl.ds(i * _L, _L)] = uk
            b0 = lax.shift_right_logical(uk, 24)
            cnts, lastm = plsc.scan_count(b0)
            plsc.addupdate_scatter(hist_v, [b0], cnts, mask=lastm)

        bstar, above = scan_hist(jnp.int32(k))
        krem = jnp.int32(k) - above
        prefix = bstar

        for shift in (16, 8, 0):
            zero_hist()

            @plsc.parallel_loop(0, nvec, unroll=unroll)
            def _(i, shift=shift, prefix=prefix):
                uk = ukey_v[pl.ds(i * _L, _L)]
                m = lax.shift_right_logical(uk, shift + 8) == prefix
                bv = lax.shift_right_logical(uk, shift) & jnp.int32(0xFF)
                cnts, lastm = plsc.scan_count(bv, m)
                plsc.addupdate_scatter(hist_v, [bv], cnts, mask=lastm)

            bstar, above = scan_hist(krem)
            krem = krem - above
            prefix = (prefix << 8) | bstar

        sbits = prefix ^ (((~prefix) >> 31) | jnp.int32(_INT_MIN))
        out_v[...] = jnp.broadcast_to(sbits, (_L,))
        pltpu.sync_copy(out_v, thr_hbm.at[row])


def _sc_thresholds(x):
    B, D = x.shape
    k = max(1, int(D * _KEEP_RATIO))
    rows_per_w = B // _NW
    mesh = plsc.VectorSubcoreMesh(core_axis_name="c", subcore_axis_name="s")
    body = functools.partial(
        _sc_thresholds_body, rows_per_w=rows_per_w, d=D, k=k, unroll=16)
    return pl.kernel(
        body,
        out_type=jax.ShapeDtypeStruct((B, _L), jnp.int32),
        mesh=mesh,
        compiler_params=pltpu.CompilerParams(needs_layout_passes=False),
        scratch_types=[
            pltpu.VMEM((2, D), jnp.int32),
            pltpu.VMEM((D,), jnp.int32),
            pltpu.VMEM((256,), jnp.int32),
            pltpu.VMEM((_L,), jnp.int32),
            pltpu.SemaphoreType.DMA,
        ],
    )(x)


def _mask_block(x_ref, t_ref, o_ref):
    x = x_ref[...]
    t = lax.bitcast_convert_type(t_ref[...][:, 0:1], jnp.float32)
    o_ref[...] = jnp.where(x >= t, x, jnp.float32(0.0))


@jax.jit
def kernel(expanded_features):
    B, D = expanded_features.shape
    x_bits = lax.bitcast_convert_type(expanded_features, jnp.int32)
    thr = _sc_thresholds(x_bits)
    block_rows = 16
    return pl.pallas_call(
        _mask_block,
        grid=(B // block_rows,),
        in_specs=[
            pl.BlockSpec((block_rows, D), lambda i: (i, 0)),
            pl.BlockSpec((block_rows, _L), lambda i: (i, 0)),
        ],
        out_specs=pl.BlockSpec((block_rows, D), lambda i: (i, 0)),
        out_shape=jax.ShapeDtypeStruct((B, D), jnp.float32),
    )(expanded_features, thr)


# speedup vs baseline: 1.0865x; 1.0865x over previous
import functools

import jax
import jax.numpy as jnp
from jax import lax
from jax.experimental import pallas as pl
from jax.experimental.pallas import tpu as pltpu
from jax.experimental.pallas import tpu_sc as plsc

_KEEP_RATIO = 0.05
_INT_MIN = -(2 ** 31)
_NC, _NS, _L = 2, 16, 16
_NW = _NC * _NS


def _sc_thresholds_body(x_hbm, thr_hbm, row_v, ukey_v, hist_v, ct_v, out_v,
                        sem, *, rows_per_w: int, d: int, k: int, unroll: int):
    cid = lax.axis_index("c")
    sid = lax.axis_index("s")
    w = sid * _NC + cid
    lanes = lax.iota(jnp.int32, _L)
    zeros16 = jnp.zeros((_L,), jnp.int32)
    lane0 = lanes == 0
    nvec = d // _L

    def scan_vreg(t, krem, above):
        rt = lax.rev(t, (0,))
        cs = plsc.cumsum(rt)
        below = jnp.sum((above + cs < krem).astype(jnp.int32))
        bin_here = jnp.int32(_L - 1) - below
        abv_here = above + jnp.sum(jnp.where(lanes < below, rt, jnp.int32(0)))
        return bin_here, abv_here

    def scan_flat(ref, nchunks, krem):
        above = jnp.int32(0)
        bstar = jnp.int32(0)
        above_b = jnp.int32(0)
        found = jnp.bool_(False)
        for j in range(nchunks - 1, -1, -1):
            t = ref[pl.ds(j * _L, _L)]
            tot = jnp.sum(t)
            bin_here, abv_here = scan_vreg(t, krem, above)
            hit = jnp.logical_and(jnp.logical_not(found), above + tot >= krem)
            bstar = jnp.where(hit, jnp.int32(j * _L) + bin_here, bstar)
            above_b = jnp.where(hit, abv_here, above_b)
            found = jnp.logical_or(found, above + tot >= krem)
            above = above + tot
        return bstar, above_b

    def scan_hist4096(krem):
        @plsc.parallel_loop(0, 4096 // _L, unroll=8)
        def _(j):
            ct = jnp.sum(hist_v[pl.ds(j * _L, _L)])
            plsc.store_scatter(ct_v, [jnp.broadcast_to(j, (_L,))],
                               jnp.broadcast_to(ct, (_L,)), mask=lane0)
        jc, above_c = scan_flat(ct_v, 256 // _L, krem)
        t = plsc.load_gather(hist_v, [jc * _L + lanes])
        bin_in, above_b = scan_vreg(t, krem, above_c)
        return (jc * _L) + bin_in, above_b

    def zero_hist(nbins):
        for j in range(nbins // _L):
            hist_v[pl.ds(j * _L, _L)] = zeros16

    nxt = pltpu.async_copy(x_hbm.at[w * rows_per_w], row_v.at[0], sem)
    for r in range(rows_per_w):
        row = w * rows_per_w + r
        nxt.wait()
        if r + 1 < rows_per_w:
            nxt = pltpu.async_copy(
                x_hbm.at[row + 1], row_v.at[(r + 1) % 2], sem)
        buf = r % 2

        zero_hist(4096)

        @plsc.parallel_loop(0, nvec, unroll=unroll)
        def _(i):
            s = row_v[buf, pl.ds(i * _L, _L)]
            uk = s ^ ((s >> 31) | jnp.int32(_INT_MIN))
            ukey_v[pl.ds(i * _L, _L)] = uk
            b0 = lax.shift_right_logical(uk, 20)
            cnts, lastm = plsc.scan_count(b0)
            plsc.addupdate_scatter(hist_v, [b0], cnts, mask=lastm)

        bstar, above = scan_hist4096(jnp.int32(k))
        krem = jnp.int32(k) - above
        prefix = bstar

        zero_hist(4096)

        @plsc.parallel_loop(0, nvec, unroll=unroll)
        def _(i, prefix=prefix):
            uk = ukey_v[pl.ds(i * _L, _L)]
            m = lax.shift_right_logical(uk, 20) == prefix
            bv = lax.shift_right_logical(uk, 8) & jnp.int32(0xFFF)
            cnts, lastm = plsc.scan_count(bv, m)
            plsc.addupdate_scatter(hist_v, [bv], cnts, mask=lastm)

        bstar, above = scan_hist4096(krem)
        krem = krem - above
        prefix = (prefix << 12) | bstar

        zero_hist(256)

        @plsc.parallel_loop(0, nvec, unroll=unroll)
        def _(i, prefix=prefix):
            uk = ukey_v[pl.ds(i * _L, _L)]
            m = lax.shift_right_logical(uk, 8) == prefix
            bv = uk & jnp.int32(0xFF)
            cnts, lastm = plsc.scan_count(bv, m)
            plsc.addupdate_scatter(hist_v, [bv], cnts, mask=lastm)

        bstar, _ = scan_flat(hist_v, 256 // _L, krem)
        prefix = (prefix << 8) | bstar

        sbits = prefix ^ (((~prefix) >> 31) | jnp.int32(_INT_MIN))
        out_v[...] = jnp.broadcast_to(sbits, (_L,))
        pltpu.sync_copy(out_v, thr_hbm.at[row])


def _sc_thresholds(x):
    B, D = x.shape
    k = max(1, int(D * _KEEP_RATIO))
    rows_per_w = B // _NW
    mesh = plsc.VectorSubcoreMesh(core_axis_name="c", subcore_axis_name="s")
    body = functools.partial(
        _sc_thresholds_body, rows_per_w=rows_per_w, d=D, k=k, unroll=8)
    return pl.kernel(
        body,
        out_type=jax.ShapeDtypeStruct((B, _L), jnp.int32),
        mesh=mesh,
        compiler_params=pltpu.CompilerParams(needs_layout_passes=False),
        scratch_types=[
            pltpu.VMEM((2, D), jnp.int32),
            pltpu.VMEM((D,), jnp.int32),
            pltpu.VMEM((4096,), jnp.int32),
            pltpu.VMEM((256,), jnp.int32),
            pltpu.VMEM((_L,), jnp.int32),
            pltpu.SemaphoreType.DMA,
        ],
    )(x)


def _mask_block(x_ref, t_ref, o_ref):
    x = x_ref[...]
    t = lax.bitcast_convert_type(t_ref[...][:, 0:1], jnp.float32)
    o_ref[...] = jnp.where(x >= t, x, jnp.float32(0.0))


@jax.jit
def kernel(expanded_features):
    B, D = expanded_features.shape
    x_bits = lax.bitcast_convert_type(expanded_features, jnp.int32)
    thr = _sc_thresholds(x_bits)
    block_rows = 16
    return pl.pallas_call(
        _mask_block,
        grid=(B // block_rows,),
        in_specs=[
            pl.BlockSpec((block_rows, D), lambda i: (i, 0)),
            pl.BlockSpec((block_rows, _L), lambda i: (i, 0)),
        ],
        out_specs=pl.BlockSpec((block_rows, D), lambda i: (i, 0)),
        out_shape=jax.ShapeDtypeStruct((B, D), jnp.float32),
    )(expanded_features, thr)


# speedup vs baseline: 1.1234x; 1.0339x over previous
import functools

import jax
import jax.numpy as jnp
from jax import lax
from jax.experimental import pallas as pl
from jax.experimental.pallas import tpu as pltpu
from jax.experimental.pallas import tpu_sc as plsc

_KEEP_RATIO = 0.05
_INT_MIN = -(2 ** 31)
_NC, _NS, _L = 2, 16, 16
_NW = _NC * _NS


def _sc_thresholds_body(x_hbm, thr_hbm, row_v, ukey_v, hist_v, ct_v, out_v,
                        sem, *, rows_per_w: int, d: int, k: int, unroll: int):
    cid = lax.axis_index("c")
    sid = lax.axis_index("s")
    w = sid * _NC + cid
    lanes = lax.iota(jnp.int32, _L)
    zeros16 = jnp.zeros((_L,), jnp.int32)
    ones = jnp.ones((_L,), jnp.int32)
    lane0 = lanes == 0
    nvec = d // _L

    def scan_vreg(t, krem, above):
        rt = lax.rev(t, (0,))
        cs = plsc.cumsum(rt)
        below = jnp.sum((above + cs < krem).astype(jnp.int32))
        bin_here = jnp.int32(_L - 1) - below
        abv_here = above + jnp.sum(jnp.where(lanes < below, rt, jnp.int32(0)))
        return bin_here, abv_here

    def scan_flat(ref, nchunks, krem, zero=False):
        above = jnp.int32(0)
        bstar = jnp.int32(0)
        above_b = jnp.int32(0)
        found = jnp.bool_(False)
        for j in range(nchunks - 1, -1, -1):
            t = ref[pl.ds(j * _L, _L)]
            if zero:
                ref[pl.ds(j * _L, _L)] = zeros16
            tot = jnp.sum(t)
            bin_here, abv_here = scan_vreg(t, krem, above)
            hit = jnp.logical_and(jnp.logical_not(found), above + tot >= krem)
            bstar = jnp.where(hit, jnp.int32(j * _L) + bin_here, bstar)
            above_b = jnp.where(hit, abv_here, above_b)
            found = jnp.logical_or(found, above + tot >= krem)
            above = above + tot
        return bstar, above_b

    def scan_hist4096(krem):
        @plsc.parallel_loop(0, 4096 // _L, unroll=8)
        def _(j):
            ct = jnp.sum(hist_v[pl.ds(j * _L, _L)])
            hist_v[pl.ds(j * _L, _L)] = zeros16
            plsc.store_scatter(ct_v, [jnp.broadcast_to(j, (_L,))],
                               jnp.broadcast_to(ct, (_L,)), mask=lane0)
        jc, above_c = scan_flat(ct_v, 256 // _L, krem)
        t = plsc.load_gather(hist_v, [jc * _L + lanes])
        bin_in, above_b = scan_vreg(t, krem, above_c)
        return (jc * _L) + bin_in, above_b

    def zero_hist(nbins):
        for j in range(nbins // _L):
            hist_v[pl.ds(j * _L, _L)] = zeros16

    zero_hist(4096)
    nxt = pltpu.async_copy(x_hbm.at[w * rows_per_w], row_v.at[0], sem)
    for r in range(rows_per_w):
        row = w * rows_per_w + r
        nxt.wait()
        if r + 1 < rows_per_w:
            nxt = pltpu.async_copy(
                x_hbm.at[row + 1], row_v.at[(r + 1) % 2], sem)
        buf = r % 2

        @plsc.parallel_loop(0, nvec, unroll=unroll)
        def _(i):
            s = row_v[buf, pl.ds(i * _L, _L)]
            uk = s ^ ((s >> 31) | jnp.int32(_INT_MIN))
            ukey_v[pl.ds(i * _L, _L)] = uk
            b0 = lax.shift_right_logical(uk, 20)
            cnts, lastm = plsc.scan_count(b0)
            plsc.addupdate_scatter(hist_v, [b0], cnts, mask=lastm)

        bstar, above = scan_hist4096(jnp.int32(k))
        krem = jnp.int32(k) - above
        prefix = bstar

        @plsc.parallel_loop(0, nvec, unroll=unroll)
        def _(i, prefix=prefix):
            uk = ukey_v[pl.ds(i * _L, _L)]
            m = lax.shift_right_logical(uk, 20) == prefix
            bv = lax.shift_right_logical(uk, 8) & jnp.int32(0xFFF)
            cnts, lastm = plsc.scan_count(bv, m)
            plsc.addupdate_scatter(hist_v, [bv], cnts, mask=lastm)

        bstar, above = scan_hist4096(krem)
        krem = krem - above
        prefix = (prefix << 12) | bstar

        @plsc.parallel_loop(0, nvec, unroll=unroll)
        def _(i, prefix=prefix):
            uk = ukey_v[pl.ds(i * _L, _L)]
            m = lax.shift_right_logical(uk, 8) == prefix
            bv = uk & jnp.int32(0xFF)
            cnts, lastm = plsc.scan_count(bv, m)
            plsc.addupdate_scatter(hist_v, [bv], cnts, mask=lastm)

        bstar, _ = scan_flat(hist_v, 256 // _L, krem, zero=True)
        prefix = (prefix << 8) | bstar

        sbits = prefix ^ (((~prefix) >> 31) | jnp.int32(_INT_MIN))
        out_v[...] = jnp.broadcast_to(sbits, (_L,))
        pltpu.sync_copy(out_v, thr_hbm.at[row])


def _sc_thresholds(x):
    B, D = x.shape
    k = max(1, int(D * _KEEP_RATIO))
    rows_per_w = B // _NW
    mesh = plsc.VectorSubcoreMesh(core_axis_name="c", subcore_axis_name="s")
    body = functools.partial(
        _sc_thresholds_body, rows_per_w=rows_per_w, d=D, k=k, unroll=8)
    return pl.kernel(
        body,
        out_type=jax.ShapeDtypeStruct((B, _L), jnp.int32),
        mesh=mesh,
        compiler_params=pltpu.CompilerParams(needs_layout_passes=False),
        scratch_types=[
            pltpu.VMEM((2, D), jnp.int32),
            pltpu.VMEM((D,), jnp.int32),
            pltpu.VMEM((4096,), jnp.int32),
            pltpu.VMEM((256,), jnp.int32),
            pltpu.VMEM((_L,), jnp.int32),
            pltpu.SemaphoreType.DMA,
        ],
    )(x)


def _mask_block(x_ref, t_ref, o_ref):
    x = x_ref[...]
    t = lax.bitcast_convert_type(t_ref[...][:, 0:1], jnp.float32)
    o_ref[...] = jnp.where(x >= t, x, jnp.float32(0.0))


@jax.jit
def kernel(expanded_features):
    B, D = expanded_features.shape
    x_bits = lax.bitcast_convert_type(expanded_features, jnp.int32)
    thr = _sc_thresholds(x_bits)
    block_rows = 16
    return pl.pallas_call(
        _mask_block,
        grid=(B // block_rows,),
        in_specs=[
            pl.BlockSpec((block_rows, D), lambda i: (i, 0)),
            pl.BlockSpec((block_rows, _L), lambda i: (i, 0)),
        ],
        out_specs=pl.BlockSpec((block_rows, D), lambda i: (i, 0)),
        out_shape=jax.ShapeDtypeStruct((B, D), jnp.float32),
    )(expanded_features, thr)
